# Initial kernel scaffold; baseline (speedup 1.0000x reference)
import jax
import jax.numpy as jnp
from jax.experimental import pallas as pl


def kernel(x, W_in, b_in, W1, g1, be1, W2, g2, be2, W3, g3, be3, W4, g4, be4):
    raise NotImplementedError("write your pallas kernel here")


# speedup vs baseline: 1.0000x; 1.0000x over previous
import jax, jax.numpy as jnp
from jax.experimental import pallas as pl

DGK = 16
NPOINTS = 2048
DOWN_NUM = 256


def _knn_idx(coor_k, coor_q, k):
    q = jax.lax.stop_gradient(coor_q).transpose(0, 2, 1)
    r = jax.lax.stop_gradient(coor_k).transpose(0, 2, 1)
    d = jnp.sum(q * q, -1)[:, :, None] - 2.0 * jnp.einsum('bqc,bkc->bqk', q, r) + jnp.sum(r * r, -1)[:, None, :]
    _, idx = jax.lax.top_k(-d, k)
    return idx


def _get_graph_feature(coor_q, x_q, coor_k, x_k, k):
    idx = _knn_idx(coor_k, coor_q, k)
    xk_t = x_k.transpose(0, 2, 1)
    feat = jax.vmap(lambda xk, ii: xk[ii])(xk_t, idx)
    feat = feat.transpose(0, 3, 1, 2)
    xq = x_q[:, :, :, None]
    feat = jnp.concatenate([feat - xq, jnp.broadcast_to(xq, feat.shape)], axis=1)
    return feat


def _group_norm(x, gamma, beta, groups=4, eps=1e-5):
    B, C, H, W = x.shape
    xg = x.reshape(B, groups, C // groups, H, W)
    m = jnp.mean(xg, axis=(2, 3, 4), keepdims=True)
    v = jnp.var(xg, axis=(2, 3, 4), keepdims=True)
    xg = (xg - m) / jnp.sqrt(v + eps)
    x = xg.reshape(B, C, H, W)
    return x * gamma[None, :, None, None] + beta[None, :, None, None]


def _conv_block(f, W, g, b):
    f = jnp.einsum('oi,bihw->bohw', W, f)
    f = _group_norm(f, g, b)
    return jnp.where(f >= 0, f, 0.2 * f)


def _fps(xyz, n_samples):
    B, N, _ = xyz.shape
    xyz = jax.lax.stop_gradient(xyz)
    dists0 = jnp.full((B, N), 1e10, dtype=xyz.dtype)
    init_last = jnp.zeros((B,), jnp.int32)
    def step(carry, _):
        dists, last = carry
        last_pt = xyz[jnp.arange(B), last]
        d = jnp.sum((xyz - last_pt[:, None, :]) ** 2, axis=-1)
        dists = jnp.minimum(dists, d)
        nxt = jnp.argmax(dists, axis=-1).astype(jnp.int32)
        return (dists, nxt), nxt
    _, rest = jax.lax.scan(step, (dists0, init_last), None, length=n_samples - 1)
    return jnp.concatenate([init_last[:, None], rest.T], axis=1)


def _fps_downsample(coor, x, num_group):
    xyz = coor.transpose(0, 2, 1)
    fi = _fps(xyz, num_group)
    combined = jnp.concatenate([coor, x], axis=1)
    newc = jax.vmap(lambda c, ii: c[:, ii])(combined, fi)
    return newc[:, :3], newc[:, 3:]


def kernel(x, W_in, b_in, W1, g1, be1, W2, g2, be2, W3, g3, be3, W4, g4, be4):
    coor = x
    f = jnp.einsum('oi,bin->bon', W_in, x) + b_in[None, :, None]
    inpc_f = f
    f = _get_graph_feature(coor, f, coor, f, DGK)
    f = _conv_block(f, W1, g1, be1)
    f = jnp.max(f, axis=-1)
    coor_q, f_q = _fps_downsample(coor, f, NPOINTS // 2)
    f = _get_graph_feature(coor_q, f_q, coor, f, DGK)
    f = _conv_block(f, W2, g2, be2)
    f = jnp.max(f, axis=-1)
    xyz1, point1 = coor_q, f
    coor = coor_q
    f = _get_graph_feature(coor, f, coor, f, DGK)
    f = _conv_block(f, W3, g3, be3)
    f = jnp.max(f, axis=-1)
    coor_q, f_q = _fps_downsample(coor, f, DOWN_NUM)
    f = _get_graph_feature(coor_q, f_q, coor, f, DGK)
    f = _conv_block(f, W4, g4, be4)
    f = jnp.max(f, axis=-1)
    return (coor_q, f, xyz1, point1, inpc_f)


# speedup vs baseline: 14.1488x; 14.1483x over previous
import functools

import jax
import jax.numpy as jnp
from jax import lax
from jax.experimental import pallas as pl
from jax.experimental.pallas import tpu as pltpu
from jax.experimental.pallas import tpu_sc as plsc

_DGK = 16
_B = 8
_N0 = 2048
_NW = 32
_NC = 2
_EPS = 1e-5
_BIG = 3.0e38


def _group_center(p, q, n_rows):
    C = p.shape[1]
    G = C // 4
    cs = jnp.sum(p, axis=0, keepdims=True) + jnp.sum(q, axis=0, keepdims=True)
    parts = []
    inv = 1.0 / (n_rows * G)
    for g in range(4):
        s = jnp.sum(cs[:, g * G:(g + 1) * G], axis=1, keepdims=True) * inv
        parts.append(jnp.broadcast_to(s, (1, G)))
    return jnp.concatenate(parts, axis=1)


def _prep1_body(x_ref, win_ref, bin_ref, wa_ref, wd_ref, f_ref, p_ref, q_ref,
                c_ref):
    xb = x_ref[0]
    f = jnp.dot(win_ref[...], xb, preferred_element_type=jnp.float32)
    f = f + bin_ref[...]
    f_ref[0] = f
    dn = (((0,), (0,)), ((), ()))
    p = lax.dot_general(f, wa_ref[...], dn, preferred_element_type=jnp.float32)
    q = lax.dot_general(f, wd_ref[...], dn, preferred_element_type=jnp.float32)
    p_ref[0] = p
    q_ref[0] = q
    c_ref[0] = _group_center(p, q, p.shape[0])


def _prep1(x, W_in, b_in, W1aT, W1dT):
    B, _, N = x.shape
    co = W1aT.shape[1]
    return pl.pallas_call(
        _prep1_body,
        grid=(B,),
        in_specs=[
            pl.BlockSpec((1, 3, N), lambda b: (b, 0, 0)),
            pl.BlockSpec((8, 3), lambda b: (0, 0)),
            pl.BlockSpec((8, 1), lambda b: (0, 0)),
            pl.BlockSpec((8, co), lambda b: (0, 0)),
            pl.BlockSpec((8, co), lambda b: (0, 0)),
        ],
        out_specs=[
            pl.BlockSpec((1, 8, N), lambda b: (b, 0, 0)),
            pl.BlockSpec((1, N, co), lambda b: (b, 0, 0)),
            pl.BlockSpec((1, N, co), lambda b: (b, 0, 0)),
            pl.BlockSpec((1, 1, co), lambda b: (b, 0, 0)),
        ],
        out_shape=[
            jax.ShapeDtypeStruct((B, 8, N), jnp.float32),
            jax.ShapeDtypeStruct((B, N, co), jnp.float32),
            jax.ShapeDtypeStruct((B, N, co), jnp.float32),
            jax.ShapeDtypeStruct((B, 1, co), jnp.float32),
        ],
    )(x, W_in, b_in.reshape(8, 1), W1aT, W1dT)


def _knn_body(nk, qc_ref, kc_ref, out_ref):
    b = pl.program_id(0)
    q = qc_ref[0]
    k = kc_ref[0]
    qk = lax.dot_general(q.astype(jnp.bfloat16), k.astype(jnp.bfloat16),
                         (((0,), (0,)), ((), ())),
                         preferred_element_type=jnp.float32)
    kk = (k[0:1, :] * k[0:1, :] + k[1:2, :] * k[1:2, :]) + k[2:3, :] * k[2:3, :]
    qq_row = (q[0:1, :] * q[0:1, :] + q[1:2, :] * q[1:2, :]) + q[2:3, :] * q[2:3, :]
    rb = q.shape[1]
    eye = (lax.broadcasted_iota(jnp.int32, (rb, rb), 0)
           == lax.broadcasted_iota(jnp.int32, (rb, rb), 1)).astype(jnp.float32)
    qq = lax.dot_general(eye, qq_row, (((1,), (1,)), ((), ())),
                         preferred_element_type=jnp.float32,
                         precision=lax.Precision.HIGHEST)
    d = (qq - 2.0 * qk) + kk
    col = lax.broadcasted_iota(jnp.int32, d.shape, 1)
    base = b * nk
    for i in range(_DGK):
        mval = jnp.min(d, axis=1, keepdims=True)
        cand = jnp.where(d == mval, col, nk)
        midx = jnp.min(cand, axis=1, keepdims=True)
        out_ref[0, :, i:i + 1] = midx + base
        d = jnp.where(col == midx, _BIG, d)


def _knn(coor_q, coor_k):
    B, _, Nq = coor_q.shape
    Nk = coor_k.shape[2]
    RB = 256
    return pl.pallas_call(
        functools.partial(_knn_body, Nk),
        grid=(B, Nq // RB),
        in_specs=[
            pl.BlockSpec((1, 3, RB), lambda b, r: (b, 0, r)),
            pl.BlockSpec((1, 3, Nk), lambda b, r: (b, 0, 0)),
        ],
        out_specs=pl.BlockSpec((1, RB, _DGK), lambda b, r: (b, r, 0)),
        out_shape=jax.ShapeDtypeStruct((B, Nq, _DGK), jnp.int32),
    )(coor_q, coor_k)


def _fps_body(nsamp, coor_ref, fig_ref, cq_ref):
    B, _, N = coor_ref.shape
    c0 = coor_ref[:, 0, :]
    c1 = coor_ref[:, 1, :]
    c2 = coor_ref[:, 2, :]
    lanesN = lax.broadcasted_iota(jnp.int32, (B, N), 1)
    lanesS = lax.broadcasted_iota(jnp.int32, (B, nsamp), 1)

    def gather_last(last):
        oh = lanesN == last
        g0 = jnp.sum(jnp.where(oh, c0, 0.0), axis=1, keepdims=True)
        g1 = jnp.sum(jnp.where(oh, c1, 0.0), axis=1, keepdims=True)
        g2 = jnp.sum(jnp.where(oh, c2, 0.0), axis=1, keepdims=True)
        return g0, g1, g2

    def step(i, carry):
        dists, last, fi, cq = carry
        g0, g1, g2 = gather_last(last)
        selc = (lanesS == i)[:, None, :]
        gvec = jnp.concatenate([g0[:, None, :], g1[:, None, :],
                                g2[:, None, :]], axis=1)
        cq = jnp.where(selc, gvec, cq)
        e0 = c0 - g0
        e1 = c1 - g1
        e2 = c2 - g2
        d = (e0 * e0 + e1 * e1) + e2 * e2
        dists = jnp.minimum(dists, d)
        mx = jnp.max(dists, axis=1, keepdims=True)
        nxt = jnp.min(jnp.where(dists == mx, lanesN, N), axis=1,
                      keepdims=True)
        fi = jnp.where(lanesS == i + 1, nxt, fi)
        return dists, nxt, fi, cq

    dists0 = jnp.full((B, N), 1e10, jnp.float32)
    last0 = jnp.zeros((B, 1), jnp.int32)
    fi0 = jnp.zeros((B, nsamp), jnp.int32)
    cq0 = jnp.zeros((B, 3, nsamp), jnp.float32)
    dists, last, fi, cq = lax.fori_loop(
        0, nsamp - 1, step, (dists0, last0, fi0, cq0))
    g0, g1, g2 = gather_last(last)
    selc = (lanesS == nsamp - 1)[:, None, :]
    gvec = jnp.concatenate([g0[:, None, :], g1[:, None, :],
                            g2[:, None, :]], axis=1)
    cq = jnp.where(selc, gvec, cq)
    boff = lax.broadcasted_iota(jnp.int32, (B, nsamp), 0) * N
    fig_ref[...] = fi + boff
    cq_ref[...] = cq


def _fps(coor, nsamp):
    B, _, N = coor.shape
    return pl.pallas_call(
        functools.partial(_fps_body, nsamp),
        out_shape=[
            jax.ShapeDtypeStruct((B, nsamp), jnp.int32),
            jax.ShapeDtypeStruct((B, 3, nsamp), jnp.float32),
        ],
    )(coor)


def _agg_sc(idx, qmap, P, Q, cvec):
    R = qmap.shape[0]
    C = P.shape[1]
    nq = R // _NW
    T = max(8, min(64, 2048 // C))
    while nq % T:
        T //= 2
    nt = nq // T
    NV = C // 16
    mesh = plsc.VectorSubcoreMesh(core_axis_name="c", subcore_axis_name="s")

    @functools.partial(
        pl.kernel,
        out_type=[
            jax.ShapeDtypeStruct((R, C), jnp.float32),
            jax.ShapeDtypeStruct((_NW, C), jnp.float32),
            jax.ShapeDtypeStruct((_NW, C), jnp.float32),
        ],
        mesh=mesh,
        scratch_types=[
            pltpu.VMEM((nq * 16,), jnp.int32),
            pltpu.VMEM((nq,), jnp.int32),
            pltpu.VMEM((nq, C), jnp.float32),
            pltpu.VMEM((T * 16, C), jnp.float32),
            pltpu.VMEM((T, C), jnp.float32),
            pltpu.VMEM((C,), jnp.float32),
            pltpu.VMEM((C,), jnp.float32),
            pltpu.VMEM((C,), jnp.float32),
            pltpu.SemaphoreType.DMA,
            pltpu.SemaphoreType.DMA,
        ],
        compiler_params=pltpu.CompilerParams(use_tc_tiling_on_sc=False),
    )
    def agg(idx_hbm, qmap_hbm, p_hbm, q_hbm, c_hbm, m_hbm, ps_hbm, pss_hbm,
            idx_v, qmap_v, qrows_v, rows_v, m_v, sacc_v, ssacc_v, cv_v,
            sem, sem2):
        wid = lax.axis_index("s") * _NC + lax.axis_index("c")
        qbase = wid * nq
        b = wid // (_NW // _B)
        pltpu.sync_copy(idx_hbm.at[pl.ds(qbase * 16, nq * 16)], idx_v)
        pltpu.sync_copy(qmap_hbm.at[pl.ds(qbase, nq)], qmap_v)
        pltpu.sync_copy(c_hbm.at[b], cv_v)
        pltpu.async_copy(q_hbm.at[qmap_v], qrows_v, sem).wait()
        zero = jnp.zeros((16,), jnp.float32)
        for j in range(NV):
            sacc_v[pl.ds(16 * j, 16)] = zero
            ssacc_v[pl.ds(16 * j, 16)] = zero
        for t in range(nt):
            pltpu.async_copy(
                p_hbm.at[idx_v.at[pl.ds(t * T * 16, T * 16)]],
                rows_v, sem2).wait()

            def qloop(qi, _):
                qrow = t * T + qi
                for j in range(NV):
                    sl = pl.ds(16 * j, 16)
                    qv = qrows_v[qrow, sl]
                    cv = cv_v[sl]
                    v = qv + rows_v[qi * 16, sl]
                    m = v
                    s = v
                    vs = v - cv
                    ss = vs * vs
                    for kk in range(1, 16):
                        v = qv + rows_v[qi * 16 + kk, sl]
                        m = jnp.maximum(m, v)
                        s = s + v
                        vs = v - cv
                        ss = ss + vs * vs
                    m_v[qi, sl] = m
                    sacc_v[sl] = sacc_v[sl] + s
                    ssacc_v[sl] = ssacc_v[sl] + ss
                return 0

            lax.fori_loop(0, T, qloop, 0)
            pltpu.sync_copy(m_v, m_hbm.at[pl.ds(qbase + t * T, T)])
        pltpu.sync_copy(sacc_v, ps_hbm.at[wid])
        pltpu.sync_copy(ssacc_v, pss_hbm.at[wid])

    return agg(idx, qmap, P, Q, cvec)


def _epilogue(M, psum, psumsq, cvec, gamma, beta, next_ws):
    B, Nq, C = M.shape
    G = C // 4
    count = G * Nq * _DGK
    nw = len(next_ws)

    def body(m_ref, ps_ref, pss_ref, c_ref, g_ref, be_ref, *rest):
        w_refs = rest[:nw]
        f_ref = rest[nw]
        o_refs = rest[nw + 1:nw + 1 + nw]
        ps = jnp.sum(ps_ref[0], axis=0, keepdims=True)
        pss = jnp.sum(pss_ref[0], axis=0, keepdims=True)
        parts_m = []
        parts_i = []
        inv_cnt = jnp.float32(1.0 / count)
        for g in range(4):
            s = jnp.sum(ps[:, g * G:(g + 1) * G], axis=1, keepdims=True)
            s2 = jnp.sum(pss[:, g * G:(g + 1) * G], axis=1, keepdims=True)
            c = c_ref[0][:, g * G:g * G + 1]
            mean = s * inv_cnt
            dm = mean - c
            var = s2 * inv_cnt - dm * dm
            inv = lax.rsqrt(var + _EPS)
            parts_m.append(jnp.broadcast_to(mean, (1, G)))
            parts_i.append(jnp.broadcast_to(inv, (1, G)))
        meanc = jnp.concatenate(parts_m, axis=1)
        invc = jnp.concatenate(parts_i, axis=1)
        y = (m_ref[0] - meanc) * invc * g_ref[...] + be_ref[...]
        f = jnp.where(y >= 0.0, y, 0.2 * y)
        f_ref[0] = f
        outs = []
        for w_ref, o_ref in zip(w_refs, o_refs):
            o = jnp.dot(f, w_ref[...], preferred_element_type=jnp.float32)
            o_ref[0] = o
            outs.append(o)
        if nw:
            cn_ref = rest[nw + 1 + nw]
            cn_ref[0] = _group_center(outs[0], outs[1], Nq)

    in_specs = [
        pl.BlockSpec((1, Nq, C), lambda b: (b, 0, 0)),
        pl.BlockSpec((1, 4, C), lambda b: (b, 0, 0)),
        pl.BlockSpec((1, 4, C), lambda b: (b, 0, 0)),
        pl.BlockSpec((1, 1, C), lambda b: (b, 0, 0)),
        pl.BlockSpec((1, C), lambda b: (0, 0)),
        pl.BlockSpec((1, C), lambda b: (0, 0)),
    ]
    out_specs = [pl.BlockSpec((1, Nq, C), lambda b: (b, 0, 0))]
    out_shape = [jax.ShapeDtypeStruct((B, Nq, C), jnp.float32)]
    args = [M, psum, psumsq, cvec, gamma.reshape(1, C), beta.reshape(1, C)]
    for w in next_ws:
        co = w.shape[1]
        in_specs.append(pl.BlockSpec((C, co), lambda b: (0, 0)))
        out_specs.append(pl.BlockSpec((1, Nq, co), lambda b: (b, 0, 0)))
        out_shape.append(jax.ShapeDtypeStruct((B, Nq, co), jnp.float32))
        args.append(w)
    if nw:
        co = next_ws[0].shape[1]
        out_specs.append(pl.BlockSpec((1, 1, co), lambda b: (b, 0, 0)))
        out_shape.append(jax.ShapeDtypeStruct((B, 1, co), jnp.float32))
    return pl.pallas_call(
        body,
        grid=(B,),
        in_specs=in_specs,
        out_specs=out_specs,
        out_shape=out_shape,
    )(*args)


def _stage(coor_q, coor_k, P, Q, qmap, cvec, gamma, beta, Nq, next_ws):
    idx = _knn(coor_q, coor_k)
    M, ps, pss = _agg_sc(idx.reshape(-1), qmap, P, Q,
                         cvec.reshape(_B, P.shape[1]))
    C = P.shape[1]
    M = M.reshape(_B, Nq, C)
    ps = ps.reshape(_B, 4, C)
    pss = pss.reshape(_B, 4, C)
    return _epilogue(M, ps, pss, cvec, gamma, beta, next_ws)


def kernel(x, W_in, b_in, W1, g1, be1, W2, g2, be2, W3, g3, be3, W4, g4, be4):
    B, _, N = x.shape
    def split(W):
        ci = W.shape[1] // 2
        Wa = W[:, :ci]
        Wd = W[:, ci:] - Wa
        return Wa.T, Wd.T

    W1aT, W1dT = split(W1)
    W2aT, W2dT = split(W2)
    W3aT, W3dT = split(W3)
    W4aT, W4dT = split(W4)

    inpc_f, P1, Q1, c1 = _prep1(x, W_in, b_in, W1aT, W1dT)
    qmap1 = jnp.arange(B * N, dtype=jnp.int32)
    _, P2, Q2, c2 = _stage(x, x, P1.reshape(B * N, 32), Q1.reshape(B * N, 32),
                           qmap1, c1, g1, be1, N, [W2aT, W2dT])

    fi1, coor_q1 = _fps(x, N // 2)

    f2, P3, Q3, c3 = _stage(coor_q1, x, P2.reshape(B * N, 64),
                            Q2.reshape(B * N, 64), fi1.reshape(-1),
                            c2, g2, be2, N // 2, [W3aT, W3dT])

    qmap3 = jnp.arange(B * (N // 2), dtype=jnp.int32)
    f3, P4, Q4, c4 = _stage(coor_q1, coor_q1, P3.reshape(B * (N // 2), 64),
                            Q3.reshape(B * (N // 2), 64), qmap3,
                            c3, g3, be3, N // 2, [W4aT, W4dT])

    fi2, coor_q2 = _fps(coor_q1, 256)

    (f4,) = _stage(coor_q2, coor_q1, P4.reshape(B * (N // 2), 128),
                   Q4.reshape(B * (N // 2), 128), fi2.reshape(-1),
                   c4, g4, be4, 256, [])

    f_out = jnp.transpose(f4, (0, 2, 1))
    point1 = jnp.transpose(f2, (0, 2, 1))
    return (coor_q2, f_out, coor_q1, point1, inpc_f)
